# Initial kernel scaffold; baseline (speedup 1.0000x reference)
import jax
import jax.numpy as jnp
from jax.experimental import pallas as pl


def kernel(h, edge_index, W1, b1, W2, b2):
    raise NotImplementedError("write your pallas kernel here")


# speedup vs baseline: 2.4529x; 2.4529x over previous
import functools

import jax
import jax.numpy as jnp
from jax import lax
from jax.experimental import pallas as pl
from jax.experimental.pallas import tpu as pltpu
from jax.experimental.pallas import tpu_sc as plsc

H = 256
L = 16
NB = H // L

_GATHER_DNUMS = lax.GatherDimensionNumbers(
    offset_dims=(), collapsed_slice_dims=(0,), start_index_map=(0,)
)


def _lane_shuffle(x, perm):
    return lax.gather(
        x, perm[:, None], _GATHER_DNUMS, slice_sizes=(1,),
        mode=lax.GatherScatterMode.PROMISE_IN_BOUNDS,
    )


def _lane_sum(x, lane):
    for sh in (8, 4, 2, 1):
        x = x + _lane_shuffle(x, (lane + sh) & (L - 1))
    return x


def _tc_body(h_ref, wa_ref, wb_ref, b1_ref, u_ref, v_ref):
    hb = h_ref[...]
    u_ref[...] = jnp.dot(hb, wa_ref[...], preferred_element_type=jnp.float32)
    v_ref[...] = (
        jnp.dot(hb, wb_ref[...], preferred_element_type=jnp.float32)
        + b1_ref[...]
    )


def _node_tables(h, waT, wbT, b1):
    n = h.shape[0]
    blk = 1000
    grid = n // blk
    return pl.pallas_call(
        _tc_body,
        grid=(grid,),
        in_specs=[
            pl.BlockSpec((blk, H), lambda i: (i, 0)),
            pl.BlockSpec((H, H), lambda i: (0, 0)),
            pl.BlockSpec((H, H), lambda i: (0, 0)),
            pl.BlockSpec((1, H), lambda i: (0, 0)),
        ],
        out_specs=[
            pl.BlockSpec((blk, H), lambda i: (i, 0)),
            pl.BlockSpec((blk, H), lambda i: (i, 0)),
        ],
        out_shape=[
            jax.ShapeDtypeStruct((n, H), jnp.float32),
            jax.ShapeDtypeStruct((n, H), jnp.float32),
        ],
    )(h, waT, wbT, b1)


def _sc_edge_kernel(e):
    info = plsc.get_sparse_core_info()
    nc, ns = info.num_cores, info.num_subcores
    nw = nc * ns
    k = 128
    nchunk = e // k
    assert e % k == 0
    per_w = (nchunk + nw - 1) // nw

    mesh = plsc.VectorSubcoreMesh(core_axis_name="c", subcore_axis_name="s")

    @functools.partial(
        pl.kernel,
        out_type=jax.ShapeDtypeStruct((e,), jnp.float32),
        mesh=mesh,
        scratch_types=[
            pltpu.VMEM((k,), jnp.int32),
            pltpu.VMEM((k,), jnp.int32),
            pltpu.VMEM((k, H), jnp.float32),
            pltpu.VMEM((k, H), jnp.float32),
            pltpu.VMEM((H,), jnp.float32),
            pltpu.VMEM((k,), jnp.float32),
            pltpu.SemaphoreType.DMA,
            pltpu.SemaphoreType.DMA,
        ],
    )
    def sc_kernel(u_hbm, v_hbm, src_hbm, dst_hbm, w2_hbm, out_hbm,
                  sidx, didx, urows, vrows, w2v, outv, sem_u, sem_v):
        wid = lax.axis_index("s") * nc + lax.axis_index("c")
        pltpu.sync_copy(w2_hbm, w2v)
        w2_regs = [w2v[pl.ds(L * j, L)] for j in range(NB)]
        lane = lax.iota(jnp.int32, L)

        def chunk_body(c, carry):
            cid = wid + c * nw

            @pl.when(cid < nchunk)
            def _():
                base = cid * k
                pltpu.sync_copy(src_hbm.at[pl.ds(base, k)], sidx)
                pltpu.sync_copy(dst_hbm.at[pl.ds(base, k)], didx)
                cp_u = pltpu.async_copy(u_hbm.at[sidx], urows, sem_u)
                cp_v = pltpu.async_copy(v_hbm.at[didx], vrows, sem_v)
                cp_u.wait()
                cp_v.wait()

                def group_body(g, carry2):
                    out_vec = jnp.zeros((L,), jnp.float32)
                    for t in range(L):
                        i = g * L + t
                        acc = jnp.zeros((L,), jnp.float32)
                        for j in range(NB):
                            uj = urows[i, pl.ds(L * j, L)]
                            vj = vrows[i, pl.ds(L * j, L)]
                            acc = acc + jnp.maximum(uj + vj, 0.0) * w2_regs[j]
                        out_vec = jnp.where(lane == t, _lane_sum(acc, lane), out_vec)
                    outv[pl.ds(g * L, L)] = out_vec
                    return carry2

                lax.fori_loop(0, k // L, group_body, 0, unroll=False)
                pltpu.sync_copy(outv, out_hbm.at[pl.ds(base, k)])

            return carry

        lax.fori_loop(0, per_w, chunk_body, 0, unroll=False)

    return sc_kernel


@jax.jit
def kernel(h, edge_index, W1, b1, W2, b2):
    waT = W1[:, :H].T
    wbT = W1[:, H:].T
    u, v = _node_tables(h, waT, wbT, b1.reshape(1, H))
    ei = edge_index.astype(jnp.int32)
    src = ei[0]
    dst = ei[1]
    e = src.shape[0]
    scores = _sc_edge_kernel(e)(u, v, src, dst, W2.reshape(H))
    return scores + b2[0]
